# baseline (speedup 1.0000x reference)
import functools

import jax
import jax.numpy as jnp
from jax import lax
from jax.experimental import pallas as pl
from jax.experimental.pallas import tpu as pltpu
from jax.experimental.pallas import tpu_sc as plsc

N, K, C = 10000, 32, 128
D_INNER, D_STATE, D_CONV, DT_RANK = 256, 16, 4, 8

_T_PRE = 1000
_T_SCAN = 200
_G = 4
_NW = 32
_GPW = 80
_NPAD = _NW * _GPW * _G


def _ln(x, g, b):
    m = jnp.mean(x, axis=-1, keepdims=True)
    xc = x - m
    v = jnp.mean(xc * xc, axis=-1, keepdims=True)
    return xc * lax.rsqrt(v + 1e-5) * g + b


def _softplus(x):
    return jnp.maximum(x, 0.0) + jnp.log(1.0 + jnp.exp(-jnp.abs(x)))


def _silu(x):
    return x / (1.0 + jnp.exp(-x))


def _dot(a, b):
    return lax.dot_general(a, b, (((1,), (0,)), ((), ())),
                           precision=lax.Precision.HIGHEST,
                           preferred_element_type=jnp.float32)


def _pre_body(feat_ref, halo_ref, wfc1, n1g, n1b, rmsw, win, convw, convb,
              wxdt, bdt, wxb, wxc,
              f_out, xc_out, z_out, dl_out, b_out, c_out):
    i = pl.program_id(0)

    def fproj(x):
        return jnp.maximum(_ln(_dot(x, wfc1[...]), n1g[...], n1b[...]), 0.0)

    def rms(x):
        return x * lax.rsqrt(jnp.mean(x * x, axis=-1, keepdims=True) + 1e-5) \
            * rmsw[...]

    f = fproj(feat_ref[...])
    f_out[...] = f
    xz = _dot(rms(f), win[...])
    xp = xz[:, :D_INNER]
    z_out[...] = xz[:, D_INNER:]

    fh = fproj(halo_ref[...])
    xph = _dot(rms(fh), win[...])[:, :D_INNER]
    xph = jnp.where(i == 0, 0.0, xph)
    ext = jnp.concatenate([xph[8 - (D_CONV - 1):], xp], axis=0)
    s = ext[0:_T_PRE] * convw[0:1, :]
    for k in range(1, D_CONV):
        s = s + ext[k:k + _T_PRE] * convw[k:k + 1, :]
    xc = _silu(s + convb[...])
    xc_out[...] = xc
    dl_out[...] = _softplus(_dot(xc, wxdt[...]) + bdt[...])
    b_out[...] = _dot(xc, wxb[...])
    c_out[...] = _dot(xc, wxc[...])


def _pre_call(feat, wfc1, n1g, n1b, rmsw, win, convw, convb, wxdt, bdt,
              wxb, wxc):
    g = N // _T_PRE
    row = lambda w: pl.BlockSpec((_T_PRE, w), lambda i: (i, 0))
    full = lambda a, b: pl.BlockSpec((a, b), lambda i: (0, 0))
    halo = pl.BlockSpec((8, C), lambda i: (jnp.maximum(i * (_T_PRE // 8) - 1, 0), 0))
    out_shapes = (
        jax.ShapeDtypeStruct((N, C), jnp.float32),
        jax.ShapeDtypeStruct((N, D_INNER), jnp.float32),
        jax.ShapeDtypeStruct((N, D_INNER), jnp.float32),
        jax.ShapeDtypeStruct((N, D_INNER), jnp.float32),
        jax.ShapeDtypeStruct((N, D_STATE), jnp.float32),
        jax.ShapeDtypeStruct((N, D_STATE), jnp.float32),
    )
    return pl.pallas_call(
        _pre_body,
        grid=(g,),
        in_specs=[
            row(C), halo,
            full(C, C), full(1, C), full(1, C), full(1, C),
            full(C, 2 * D_INNER), full(D_CONV, D_INNER), full(1, D_INNER),
            full(D_INNER, D_INNER), full(1, D_INNER),
            full(D_INNER, D_STATE), full(D_INNER, D_STATE),
        ],
        out_specs=(row(C), row(D_INNER), row(D_INNER), row(D_INNER),
                   row(D_STATE), row(D_STATE)),
        out_shape=out_shapes,
    )(feat, feat, wfc1, n1g, n1b, rmsw, win, convw, convb, wxdt, bdt,
      wxb, wxc)


def _scan_body(dl_ref, b_ref, c_ref, xc_ref, z_ref, f_ref, at_ref, dvec_ref,
               wout_ref, nag_ref, nab_ref, fm_out, h_c, da_s, dbx_s, hist_s):
    i = pl.program_id(0)

    @pl.when(i == 0)
    def _():
        h_c[...] = jnp.zeros_like(h_c)

    dl = dl_ref[...]
    da_s[...] = jnp.exp(dl[:, None, :] * at_ref[...][None])
    dbx_s[...] = (dl * xc_ref[...])[:, None, :] * b_ref[...][:, :, None]

    def step(t, h):
        h = da_s[pl.ds(t, 1)] * h + dbx_s[pl.ds(t, 1)]
        hist_s[pl.ds(t, 1)] = h
        return h

    h = lax.fori_loop(0, _T_SCAN, step, h_c[...])
    h_c[...] = h

    ys = jnp.sum(hist_s[...] * c_ref[...][:, :, None], axis=1)
    y = (ys + xc_ref[...] * dvec_ref[...]) * _silu(z_ref[...])
    m = f_ref[...] + _dot(y, wout_ref[...])
    fm_out[...] = _ln(m, nag_ref[...], nab_ref[...])


def _scan_call(delta, bm, cm, xc, z, f, at, dvec, wout, nag, nab):
    g = N // _T_SCAN
    row = lambda w: pl.BlockSpec((_T_SCAN, w), lambda i: (i, 0))
    full = lambda a, b: pl.BlockSpec((a, b), lambda i: (0, 0))
    return pl.pallas_call(
        _scan_body,
        grid=(g,),
        in_specs=[row(D_INNER), row(D_STATE), row(D_STATE), row(D_INNER),
                  row(D_INNER), row(C),
                  full(D_STATE, D_INNER), full(1, D_INNER),
                  full(D_INNER, C), full(1, C), full(1, C)],
        out_specs=row(C),
        out_shape=jax.ShapeDtypeStruct((N, C), jnp.float32),
        scratch_shapes=[
            pltpu.VMEM((1, D_STATE, D_INNER), jnp.float32),
            pltpu.VMEM((_T_SCAN, D_STATE, D_INNER), jnp.float32),
            pltpu.VMEM((_T_SCAN, D_STATE, D_INNER), jnp.float32),
            pltpu.VMEM((_T_SCAN, D_STATE, D_INNER), jnp.float32),
        ],
    )(delta, bm, cm, xc, z, f, at, dvec, wout, nag, nab)


def _gather_body(f_hbm, idxf_hbm, gaussf_hbm, out_hbm, idx_v, w_v, rows_v,
                 out_v, lsem, gsem):
    wid = lax.axis_index("s") * 2 + lax.axis_index("c")
    gk = _G * K
    base = wid * (_GPW * gk)
    cp_i = pltpu.async_copy(idxf_hbm.at[pl.ds(base, _GPW * gk)], idx_v, lsem)
    cp_w = pltpu.async_copy(gaussf_hbm.at[pl.ds(base, _GPW * gk)], w_v, lsem)
    cp_i.wait()
    cp_w.wait()

    def body(gi, carry):
        idx_sl = idx_v.at[pl.ds(gi * gk, gk)]
        pltpu.async_copy(f_hbm.at[idx_sl], rows_v, gsem).wait()
        for r in range(_G):
            woff = gi * gk + r * K
            wvecs = [w_v[pl.ds(woff + 16 * j, 16)] for j in range(K // 16)]
            ws = [wvecs[k // 16][k % 16] for k in range(K)]
            for c8 in range(C // 16):
                parts = [jnp.zeros((16,), jnp.float32) for _ in range(4)]
                for k in range(K):
                    v = rows_v[r * K + k, pl.ds(c8 * 16, 16)]
                    parts[k % 4] = parts[k % 4] + ws[k] * v
                out_v[gi * _G + r, pl.ds(c8 * 16, 16)] = \
                    (parts[0] + parts[1]) + (parts[2] + parts[3])
        return carry

    lax.fori_loop(0, _GPW, body, 0)
    pltpu.sync_copy(out_v, out_hbm.at[pl.ds(wid * (_GPW * _G), _GPW * _G)])


def _gather_call(f, idx_flat, gauss_flat):
    mesh = plsc.VectorSubcoreMesh(core_axis_name="c", subcore_axis_name="s")
    kfn = functools.partial(
        pl.kernel,
        mesh=mesh,
        out_type=jax.ShapeDtypeStruct((_NPAD, C), jnp.float32),
        scratch_types=[
            pltpu.VMEM((_GPW * _G * K,), jnp.int32),
            pltpu.VMEM((_GPW * _G * K,), jnp.float32),
            pltpu.VMEM((_G * K, C), jnp.float32),
            pltpu.VMEM((_GPW * _G, C), jnp.float32),
            pltpu.SemaphoreType.DMA,
            pltpu.SemaphoreType.DMA,
        ],
    )(_gather_body)
    return kfn(f, idx_flat, gauss_flat)


def _post_body(fm_ref, cb_ref, feat_ref, nag, nab, wla1a, wla1b, bla1,
               lag, lab, wla2, bla2, n2g, n2b, wfc3, n3g, n3b, out_ref):
    comb = _ln(cb_ref[...], nag[...], nab[...])
    h = _dot(fm_ref[...], wla1a[...]) + _dot(comb, wla1b[...]) + bla1[...]
    h = jnp.maximum(_ln(h, lag[...], lab[...]), 0.0)
    res = _dot(h, wla2[...]) + bla2[...]
    f2 = jnp.maximum(_ln(res, n2g[...], n2b[...]), 0.0)
    f3 = _ln(_dot(f2, wfc3[...]), n3g[...], n3b[...])
    out_ref[...] = jnp.maximum(feat_ref[...] + f3, 0.0)


def _post_call(fm, cb, feat, nag, nab, wla1a, wla1b, bla1, lag, lab,
               wla2, bla2, n2g, n2b, wfc3, n3g, n3b):
    g = N // _T_PRE
    row = pl.BlockSpec((_T_PRE, C), lambda i: (i, 0))
    full = lambda a, b: pl.BlockSpec((a, b), lambda i: (0, 0))
    return pl.pallas_call(
        _post_body,
        grid=(g,),
        in_specs=[row, row, row,
                  full(1, C), full(1, C),
                  full(C, C), full(C, C), full(1, C),
                  full(1, C), full(1, C),
                  full(C, C), full(1, C), full(1, C), full(1, C),
                  full(C, C), full(1, C), full(1, C)],
        out_specs=row,
        out_shape=jax.ShapeDtypeStruct((N, C), jnp.float32),
    )(fm, cb, feat, nag, nab, wla1a, wla1b, bla1, lag, lab, wla2, bla2,
      n2g, n2b, wfc3, n3g, n3b)


def kernel(feat, coord, reference_index, gauss_dist, params):
    p = params
    mp = p['mamba']
    r1 = lambda a: a.reshape(1, -1)

    at = (-jnp.exp(mp['A_log'])).T
    wxdt = mp['W_x'][:, :DT_RANK] @ mp['W_dt']
    wxb = mp['W_x'][:, DT_RANK:DT_RANK + D_STATE]
    wxc = mp['W_x'][:, DT_RANK + D_STATE:]
    convw = mp['conv_w'].T

    f, xc, z, delta, bm, cm = _pre_call(
        feat, p['W_fc1'], r1(p['n1_g']), r1(p['n1_b']), r1(mp['rms_w']),
        mp['W_in'], convw, r1(mp['conv_b']), wxdt, r1(mp['b_dt']),
        wxb, wxc)

    npadk = (_NPAD - N) * K
    idx_flat = jnp.concatenate(
        [reference_index.reshape(-1), jnp.zeros((npadk,), jnp.int32)])
    gauss_flat = jnp.concatenate(
        [gauss_dist.reshape(-1), jnp.zeros((npadk,), jnp.float32)])
    comb = _gather_call(f, idx_flat, gauss_flat)[:N]

    fm = _scan_call(delta, bm, cm, xc, z, f, at, r1(mp['D']),
                    mp['W_out'], r1(p['na_g']), r1(p['na_b']))

    return _post_call(
        fm, comb, feat, r1(p['na_g']), r1(p['na_b']),
        p['W_la1'][:C], p['W_la1'][C:], r1(p['b_la1']),
        r1(p['la_g']), r1(p['la_b']), p['W_la2'], r1(p['b_la2']),
        r1(p['n2_g']), r1(p['n2_b']), p['W_fc3'], r1(p['n3_g']),
        r1(p['n3_b']))


# speedup vs baseline: 1.2183x; 1.2183x over previous
import functools

import jax
import jax.numpy as jnp
from jax import lax
from jax.experimental import pallas as pl
from jax.experimental.pallas import tpu as pltpu
from jax.experimental.pallas import tpu_sc as plsc

N, K, C = 10000, 32, 128
D_INNER, D_STATE, D_CONV, DT_RANK = 256, 16, 4, 8

_T_PRE = 1000
_T_SCAN = 200
_G = 4
_NW = 32
_GH = 120
_GL = 40
_NPAD = 16 * (_GH + _GL) * _G
_HEAVY_CORE = 0


def _ln(x, g, b):
    m = jnp.mean(x, axis=-1, keepdims=True)
    xc = x - m
    v = jnp.mean(xc * xc, axis=-1, keepdims=True)
    return xc * lax.rsqrt(v + 1e-5) * g + b


def _softplus(x):
    return jnp.maximum(x, 0.0) + jnp.log(1.0 + jnp.exp(-jnp.abs(x)))


def _silu(x):
    return x / (1.0 + jnp.exp(-x))


def _dot(a, b):
    return lax.dot_general(a, b, (((1,), (0,)), ((), ())),
                           precision=lax.Precision.HIGHEST,
                           preferred_element_type=jnp.float32)


def _pre_body(feat_ref, halo_ref, wfc1, n1g, n1b, rmsw, win, convw, convb,
              wxdt, bdt, wxb, wxc,
              f_out, xc_out, z_out, dl_out, b_out, c_out):
    i = pl.program_id(0)

    def fproj(x):
        return jnp.maximum(_ln(_dot(x, wfc1[...]), n1g[...], n1b[...]), 0.0)

    def rms(x):
        return x * lax.rsqrt(jnp.mean(x * x, axis=-1, keepdims=True) + 1e-5) \
            * rmsw[...]

    f = fproj(feat_ref[...])
    f_out[...] = f
    xz = _dot(rms(f), win[...])
    xp = xz[:, :D_INNER]
    z_out[...] = xz[:, D_INNER:]

    fh = fproj(halo_ref[...])
    xph = _dot(rms(fh), win[...])[:, :D_INNER]
    xph = jnp.where(i == 0, 0.0, xph)
    ext = jnp.concatenate([xph[8 - (D_CONV - 1):], xp], axis=0)
    s = ext[0:_T_PRE] * convw[0:1, :]
    for k in range(1, D_CONV):
        s = s + ext[k:k + _T_PRE] * convw[k:k + 1, :]
    xc = _silu(s + convb[...])
    xc_out[...] = xc
    dl_out[...] = _softplus(_dot(xc, wxdt[...]) + bdt[...])
    b_out[...] = _dot(xc, wxb[...])
    c_out[...] = _dot(xc, wxc[...])


def _pre_call(feat, wfc1, n1g, n1b, rmsw, win, convw, convb, wxdt, bdt,
              wxb, wxc):
    g = N // _T_PRE
    row = lambda w: pl.BlockSpec((_T_PRE, w), lambda i: (i, 0))
    full = lambda a, b: pl.BlockSpec((a, b), lambda i: (0, 0))
    halo = pl.BlockSpec((8, C), lambda i: (jnp.maximum(i * (_T_PRE // 8) - 1, 0), 0))
    out_shapes = (
        jax.ShapeDtypeStruct((N, C), jnp.float32),
        jax.ShapeDtypeStruct((N, D_INNER), jnp.float32),
        jax.ShapeDtypeStruct((N, D_INNER), jnp.float32),
        jax.ShapeDtypeStruct((N, D_INNER), jnp.float32),
        jax.ShapeDtypeStruct((N, D_STATE), jnp.float32),
        jax.ShapeDtypeStruct((N, D_STATE), jnp.float32),
    )
    return pl.pallas_call(
        _pre_body,
        grid=(g,),
        in_specs=[
            row(C), halo,
            full(C, C), full(1, C), full(1, C), full(1, C),
            full(C, 2 * D_INNER), full(D_CONV, D_INNER), full(1, D_INNER),
            full(D_INNER, D_INNER), full(1, D_INNER),
            full(D_INNER, D_STATE), full(D_INNER, D_STATE),
        ],
        out_specs=(row(C), row(D_INNER), row(D_INNER), row(D_INNER),
                   row(D_STATE), row(D_STATE)),
        out_shape=out_shapes,
    )(feat, feat, wfc1, n1g, n1b, rmsw, win, convw, convb, wxdt, bdt,
      wxb, wxc)


def _scan_body(dl_ref, b_ref, c_ref, xc_ref, z_ref, f_ref, at_ref, dvec_ref,
               wout_ref, nag_ref, nab_ref, fm_out, h_c, da_s, dbx_s, hist_s):
    i = pl.program_id(0)

    @pl.when(i == 0)
    def _():
        h_c[...] = jnp.zeros_like(h_c)

    dl = dl_ref[...]
    da_s[...] = jnp.exp(dl[:, None, :] * at_ref[...][None])
    dbx_s[...] = (dl * xc_ref[...])[:, None, :] * b_ref[...][:, :, None]

    def step(t, h):
        h = da_s[pl.ds(t, 1)] * h + dbx_s[pl.ds(t, 1)]
        hist_s[pl.ds(t, 1)] = h
        return h

    h = lax.fori_loop(0, _T_SCAN, step, h_c[...])
    h_c[...] = h

    ys = jnp.sum(hist_s[...] * c_ref[...][:, :, None], axis=1)
    y = (ys + xc_ref[...] * dvec_ref[...]) * _silu(z_ref[...])
    m = f_ref[...] + _dot(y, wout_ref[...])
    fm_out[...] = _ln(m, nag_ref[...], nab_ref[...])


def _scan_call(delta, bm, cm, xc, z, f, at, dvec, wout, nag, nab):
    g = N // _T_SCAN
    row = lambda w: pl.BlockSpec((_T_SCAN, w), lambda i: (i, 0))
    full = lambda a, b: pl.BlockSpec((a, b), lambda i: (0, 0))
    return pl.pallas_call(
        _scan_body,
        grid=(g,),
        in_specs=[row(D_INNER), row(D_STATE), row(D_STATE), row(D_INNER),
                  row(D_INNER), row(C),
                  full(D_STATE, D_INNER), full(1, D_INNER),
                  full(D_INNER, C), full(1, C), full(1, C)],
        out_specs=row(C),
        out_shape=jax.ShapeDtypeStruct((N, C), jnp.float32),
        scratch_shapes=[
            pltpu.VMEM((1, D_STATE, D_INNER), jnp.float32),
            pltpu.VMEM((_T_SCAN, D_STATE, D_INNER), jnp.float32),
            pltpu.VMEM((_T_SCAN, D_STATE, D_INNER), jnp.float32),
            pltpu.VMEM((_T_SCAN, D_STATE, D_INNER), jnp.float32),
        ],
    )(delta, bm, cm, xc, z, f, at, dvec, wout, nag, nab)


def _gather_body(f_hbm, idxf_hbm, gaussf_hbm, out_hbm, idx_v, w_v, rows_v,
                 out_v, lsem, gsem):
    c = lax.axis_index("c")
    s = lax.axis_index("s")
    gk = _G * K
    heavy = c == _HEAVY_CORE
    base_g = jnp.where(heavy, s * _GH, 16 * _GH + s * _GL)
    ng = jnp.where(heavy, _GH, _GL)
    base = base_g * gk
    cp_i = pltpu.async_copy(idxf_hbm.at[pl.ds(base, _GH * gk)], idx_v, lsem)
    cp_w = pltpu.async_copy(gaussf_hbm.at[pl.ds(base, _GH * gk)], w_v, lsem)
    cp_i.wait()
    cp_w.wait()

    def fire(gi, buf):
        idx_sl = idx_v.at[pl.ds(gi * gk, gk)]
        return pltpu.async_copy(f_hbm.at[idx_sl], rows_v.at[buf], gsem)

    fire(0, 0)
    fire(1, 1)

    def body(gi, carry):
        buf = lax.rem(gi, 2)
        pltpu.make_async_copy(f_hbm.at[idx_v.at[pl.ds(0, gk)]],
                              rows_v.at[buf], gsem).wait()
        for r in range(_G):
            woff = gi * gk + r * K
            wvecs = [w_v[pl.ds(woff + 16 * j, 16)] for j in range(K // 16)]
            ws = [wvecs[k // 16][k % 16] for k in range(K)]
            for c8 in range(C // 16):
                parts = [jnp.zeros((16,), jnp.float32) for _ in range(4)]
                for k in range(K):
                    v = rows_v[buf, r * K + k, pl.ds(c8 * 16, 16)]
                    parts[k % 4] = parts[k % 4] + ws[k] * v
                out_v[gi * _G + r, pl.ds(c8 * 16, 16)] = \
                    (parts[0] + parts[1]) + (parts[2] + parts[3])
        @pl.when(gi + 2 < ng)
        def _():
            fire(gi + 2, buf)
        return carry

    lax.fori_loop(0, ng, body, 0)

    @pl.when(heavy)
    def _():
        pltpu.sync_copy(out_v, out_hbm.at[pl.ds(base_g * _G, _GH * _G)])

    @pl.when(jnp.logical_not(heavy))
    def _():
        pltpu.sync_copy(out_v.at[pl.ds(0, _GL * _G)],
                        out_hbm.at[pl.ds(base_g * _G, _GL * _G)])


def _gather_call(f, idx_flat, gauss_flat):
    mesh = plsc.VectorSubcoreMesh(core_axis_name="c", subcore_axis_name="s")
    kfn = functools.partial(
        pl.kernel,
        mesh=mesh,
        out_type=jax.ShapeDtypeStruct((_NPAD, C), jnp.float32),
        scratch_types=[
            pltpu.VMEM((_GH * _G * K,), jnp.int32),
            pltpu.VMEM((_GH * _G * K,), jnp.float32),
            pltpu.VMEM((2, _G * K, C), jnp.float32),
            pltpu.VMEM((_GH * _G, C), jnp.float32),
            pltpu.SemaphoreType.DMA,
            pltpu.SemaphoreType.DMA,
        ],
    )(_gather_body)
    return kfn(f, idx_flat, gauss_flat)


def _post_body(fm_ref, cb_ref, feat_ref, nag, nab, wla1a, wla1b, bla1,
               lag, lab, wla2, bla2, n2g, n2b, wfc3, n3g, n3b, out_ref):
    comb = _ln(cb_ref[...], nag[...], nab[...])
    h = _dot(fm_ref[...], wla1a[...]) + _dot(comb, wla1b[...]) + bla1[...]
    h = jnp.maximum(_ln(h, lag[...], lab[...]), 0.0)
    res = _dot(h, wla2[...]) + bla2[...]
    f2 = jnp.maximum(_ln(res, n2g[...], n2b[...]), 0.0)
    f3 = _ln(_dot(f2, wfc3[...]), n3g[...], n3b[...])
    out_ref[...] = jnp.maximum(feat_ref[...] + f3, 0.0)


def _post_call(fm, cb, feat, nag, nab, wla1a, wla1b, bla1, lag, lab,
               wla2, bla2, n2g, n2b, wfc3, n3g, n3b):
    g = N // _T_PRE
    row = pl.BlockSpec((_T_PRE, C), lambda i: (i, 0))
    full = lambda a, b: pl.BlockSpec((a, b), lambda i: (0, 0))
    return pl.pallas_call(
        _post_body,
        grid=(g,),
        in_specs=[row, row, row,
                  full(1, C), full(1, C),
                  full(C, C), full(C, C), full(1, C),
                  full(1, C), full(1, C),
                  full(C, C), full(1, C), full(1, C), full(1, C),
                  full(C, C), full(1, C), full(1, C)],
        out_specs=row,
        out_shape=jax.ShapeDtypeStruct((N, C), jnp.float32),
    )(fm, cb, feat, nag, nab, wla1a, wla1b, bla1, lag, lab, wla2, bla2,
      n2g, n2b, wfc3, n3g, n3b)


def kernel(feat, coord, reference_index, gauss_dist, params):
    p = params
    mp = p['mamba']
    r1 = lambda a: a.reshape(1, -1)

    at = (-jnp.exp(mp['A_log'])).T
    wxdt = mp['W_x'][:, :DT_RANK] @ mp['W_dt']
    wxb = mp['W_x'][:, DT_RANK:DT_RANK + D_STATE]
    wxc = mp['W_x'][:, DT_RANK + D_STATE:]
    convw = mp['conv_w'].T

    f, xc, z, delta, bm, cm = _pre_call(
        feat, p['W_fc1'], r1(p['n1_g']), r1(p['n1_b']), r1(mp['rms_w']),
        mp['W_in'], convw, r1(mp['conv_b']), wxdt, r1(mp['b_dt']),
        wxb, wxc)

    npadk = (_NPAD - N) * K + (_GH - _GL) * _G * K
    idx_flat = jnp.concatenate(
        [reference_index.reshape(-1), jnp.zeros((npadk,), jnp.int32)])
    gauss_flat = jnp.concatenate(
        [gauss_dist.reshape(-1), jnp.zeros((npadk,), jnp.float32)])
    comb = _gather_call(f, idx_flat, gauss_flat)[:N]

    fm = _scan_call(delta, bm, cm, xc, z, f, at, r1(mp['D']),
                    mp['W_out'], r1(p['na_g']), r1(p['na_b']))

    return _post_call(
        fm, comb, feat, r1(p['na_g']), r1(p['na_b']),
        p['W_la1'][:C], p['W_la1'][C:], r1(p['b_la1']),
        r1(p['la_g']), r1(p['la_b']), p['W_la2'], r1(p['b_la2']),
        r1(p['n2_g']), r1(p['n2_b']), p['W_fc3'], r1(p['n3_g']),
        r1(p['n3_b']))


# speedup vs baseline: 1.2566x; 1.0315x over previous
import functools

import jax
import jax.numpy as jnp
from jax import lax
from jax.experimental import pallas as pl
from jax.experimental.pallas import tpu as pltpu
from jax.experimental.pallas import tpu_sc as plsc

N, K, C = 10000, 32, 128
D_INNER, D_STATE, D_CONV, DT_RANK = 256, 16, 4, 8

_T_PRE = 1000
_T_SCAN = 200
_G = 4
_NW = 32
_GH = 120
_GL = 40
_NPAD = 16 * (_GH + _GL) * _G
_HEAVY_CORE = 1


def _ln(x, g, b):
    m = jnp.mean(x, axis=-1, keepdims=True)
    xc = x - m
    v = jnp.mean(xc * xc, axis=-1, keepdims=True)
    return xc * lax.rsqrt(v + 1e-5) * g + b


def _softplus(x):
    return jnp.maximum(x, 0.0) + jnp.log(1.0 + jnp.exp(-jnp.abs(x)))


def _silu(x):
    return x / (1.0 + jnp.exp(-x))


def _dot(a, b):
    return lax.dot_general(a, b, (((1,), (0,)), ((), ())),
                           precision=lax.Precision.HIGHEST,
                           preferred_element_type=jnp.float32)


def _pre_body(feat_ref, halo_ref, wfc1, n1g, n1b, rmsw, win, convw, convb,
              wxdt, bdt, wxb, wxc,
              f_out, xc_out, z_out, dl_out, b_out, c_out):
    i = pl.program_id(0)

    def fproj(x):
        return jnp.maximum(_ln(_dot(x, wfc1[...]), n1g[...], n1b[...]), 0.0)

    def rms(x):
        return x * lax.rsqrt(jnp.mean(x * x, axis=-1, keepdims=True) + 1e-5) \
            * rmsw[...]

    f = fproj(feat_ref[...])
    f_out[...] = f
    xz = _dot(rms(f), win[...])
    xp = xz[:, :D_INNER]
    z_out[...] = xz[:, D_INNER:]

    fh = fproj(halo_ref[...])
    xph = _dot(rms(fh), win[...])[:, :D_INNER]
    xph = jnp.where(i == 0, 0.0, xph)
    ext = jnp.concatenate([xph[8 - (D_CONV - 1):], xp], axis=0)
    s = ext[0:_T_PRE] * convw[0:1, :]
    for k in range(1, D_CONV):
        s = s + ext[k:k + _T_PRE] * convw[k:k + 1, :]
    xc = _silu(s + convb[...])
    xc_out[...] = xc
    dl_out[...] = _softplus(_dot(xc, wxdt[...]) + bdt[...])
    b_out[...] = _dot(xc, wxb[...])
    c_out[...] = _dot(xc, wxc[...])


def _pre_call(feat, wfc1, n1g, n1b, rmsw, win, convw, convb, wxdt, bdt,
              wxb, wxc):
    g = N // _T_PRE
    row = lambda w: pl.BlockSpec((_T_PRE, w), lambda i: (i, 0))
    full = lambda a, b: pl.BlockSpec((a, b), lambda i: (0, 0))
    halo = pl.BlockSpec((8, C), lambda i: (jnp.maximum(i * (_T_PRE // 8) - 1, 0), 0))
    out_shapes = (
        jax.ShapeDtypeStruct((N, C), jnp.float32),
        jax.ShapeDtypeStruct((N, D_INNER), jnp.float32),
        jax.ShapeDtypeStruct((N, D_INNER), jnp.float32),
        jax.ShapeDtypeStruct((N, D_INNER), jnp.float32),
        jax.ShapeDtypeStruct((N, D_STATE), jnp.float32),
        jax.ShapeDtypeStruct((N, D_STATE), jnp.float32),
    )
    return pl.pallas_call(
        _pre_body,
        grid=(g,),
        in_specs=[
            row(C), halo,
            full(C, C), full(1, C), full(1, C), full(1, C),
            full(C, 2 * D_INNER), full(D_CONV, D_INNER), full(1, D_INNER),
            full(D_INNER, D_INNER), full(1, D_INNER),
            full(D_INNER, D_STATE), full(D_INNER, D_STATE),
        ],
        out_specs=(row(C), row(D_INNER), row(D_INNER), row(D_INNER),
                   row(D_STATE), row(D_STATE)),
        out_shape=out_shapes,
    )(feat, feat, wfc1, n1g, n1b, rmsw, win, convw, convb, wxdt, bdt,
      wxb, wxc)


def _scan_body(dl_ref, b_ref, c_ref, xc_ref, z_ref, f_ref, at_ref, dvec_ref,
               wout_ref, nag_ref, nab_ref, fm_out, h_c, da_s, dbx_s, hist_s):
    i = pl.program_id(0)

    @pl.when(i == 0)
    def _():
        h_c[...] = jnp.zeros_like(h_c)

    dl = dl_ref[...]
    da_s[...] = jnp.exp(dl[:, None, :] * at_ref[...][None])
    dbx_s[...] = (dl * xc_ref[...])[:, None, :] * b_ref[...][:, :, None]

    def step(t, h):
        h = da_s[pl.ds(t, 1)] * h + dbx_s[pl.ds(t, 1)]
        hist_s[pl.ds(t, 1)] = h
        return h

    h = lax.fori_loop(0, _T_SCAN, step, h_c[...])
    h_c[...] = h

    ys = jnp.sum(hist_s[...] * c_ref[...][:, :, None], axis=1)
    y = (ys + xc_ref[...] * dvec_ref[...]) * _silu(z_ref[...])
    m = f_ref[...] + _dot(y, wout_ref[...])
    fm_out[...] = _ln(m, nag_ref[...], nab_ref[...])


def _scan_call(delta, bm, cm, xc, z, f, at, dvec, wout, nag, nab):
    g = N // _T_SCAN
    row = lambda w: pl.BlockSpec((_T_SCAN, w), lambda i: (i, 0))
    full = lambda a, b: pl.BlockSpec((a, b), lambda i: (0, 0))
    return pl.pallas_call(
        _scan_body,
        grid=(g,),
        in_specs=[row(D_INNER), row(D_STATE), row(D_STATE), row(D_INNER),
                  row(D_INNER), row(C),
                  full(D_STATE, D_INNER), full(1, D_INNER),
                  full(D_INNER, C), full(1, C), full(1, C)],
        out_specs=row(C),
        out_shape=jax.ShapeDtypeStruct((N, C), jnp.float32),
        scratch_shapes=[
            pltpu.VMEM((1, D_STATE, D_INNER), jnp.float32),
            pltpu.VMEM((_T_SCAN, D_STATE, D_INNER), jnp.float32),
            pltpu.VMEM((_T_SCAN, D_STATE, D_INNER), jnp.float32),
            pltpu.VMEM((_T_SCAN, D_STATE, D_INNER), jnp.float32),
        ],
    )(delta, bm, cm, xc, z, f, at, dvec, wout, nag, nab)


def _gather_body(f_hbm, idxf_hbm, gaussf_hbm, out_hbm, idx_v, w_v, rows_v,
                 out_v, lsem, gsem):
    c = lax.axis_index("c")
    s = lax.axis_index("s")
    gk = _G * K
    heavy = c == _HEAVY_CORE
    base_g = jnp.where(heavy, s * _GH, 16 * _GH + s * _GL)
    ng = jnp.where(heavy, _GH, _GL)
    base = base_g * gk
    cp_i = pltpu.async_copy(idxf_hbm.at[pl.ds(base, _GH * gk)], idx_v, lsem)
    cp_w = pltpu.async_copy(gaussf_hbm.at[pl.ds(base, _GH * gk)], w_v, lsem)
    cp_i.wait()
    cp_w.wait()

    def fire(gi, buf):
        idx_sl = idx_v.at[pl.ds(gi * gk, gk)]
        return pltpu.async_copy(f_hbm.at[idx_sl], rows_v.at[buf], gsem)

    fire(0, 0)
    fire(1, 1)

    def body(gi, carry):
        buf = lax.rem(gi, 2)
        pltpu.make_async_copy(f_hbm.at[idx_v.at[pl.ds(0, gk)]],
                              rows_v.at[buf], gsem).wait()
        for r in range(_G):
            woff = gi * gk + r * K
            wvecs = [w_v[pl.ds(woff + 16 * j, 16)] for j in range(K // 16)]
            ws = [wvecs[k // 16][k % 16] for k in range(K)]
            for c8 in range(C // 16):
                parts = [jnp.zeros((16,), jnp.float32) for _ in range(4)]
                for k in range(K):
                    v = rows_v[buf, r * K + k, pl.ds(c8 * 16, 16)]
                    parts[k % 4] = parts[k % 4] + ws[k] * v
                out_v[gi * _G + r, pl.ds(c8 * 16, 16)] = \
                    (parts[0] + parts[1]) + (parts[2] + parts[3])
        @pl.when(gi + 2 < ng)
        def _():
            fire(gi + 2, buf)
        return carry

    lax.fori_loop(0, ng, body, 0)

    @pl.when(heavy)
    def _():
        pltpu.sync_copy(out_v, out_hbm.at[pl.ds(base_g * _G, _GH * _G)])

    @pl.when(jnp.logical_not(heavy))
    def _():
        pltpu.sync_copy(out_v.at[pl.ds(0, _GL * _G)],
                        out_hbm.at[pl.ds(base_g * _G, _GL * _G)])


def _gather_call(f, idx_flat, gauss_flat):
    mesh = plsc.VectorSubcoreMesh(core_axis_name="c", subcore_axis_name="s")
    kfn = functools.partial(
        pl.kernel,
        mesh=mesh,
        out_type=jax.ShapeDtypeStruct((_NPAD, C), jnp.float32),
        scratch_types=[
            pltpu.VMEM((_GH * _G * K,), jnp.int32),
            pltpu.VMEM((_GH * _G * K,), jnp.float32),
            pltpu.VMEM((2, _G * K, C), jnp.float32),
            pltpu.VMEM((_GH * _G, C), jnp.float32),
            pltpu.SemaphoreType.DMA,
            pltpu.SemaphoreType.DMA,
        ],
    )(_gather_body)
    return kfn(f, idx_flat, gauss_flat)


def _post_body(fm_ref, cb_ref, feat_ref, nag, nab, wla1a, wla1b, bla1,
               lag, lab, wla2, bla2, n2g, n2b, wfc3, n3g, n3b, out_ref):
    comb = _ln(cb_ref[...], nag[...], nab[...])
    h = _dot(fm_ref[...], wla1a[...]) + _dot(comb, wla1b[...]) + bla1[...]
    h = jnp.maximum(_ln(h, lag[...], lab[...]), 0.0)
    res = _dot(h, wla2[...]) + bla2[...]
    f2 = jnp.maximum(_ln(res, n2g[...], n2b[...]), 0.0)
    f3 = _ln(_dot(f2, wfc3[...]), n3g[...], n3b[...])
    out_ref[...] = jnp.maximum(feat_ref[...] + f3, 0.0)


def _post_call(fm, cb, feat, nag, nab, wla1a, wla1b, bla1, lag, lab,
               wla2, bla2, n2g, n2b, wfc3, n3g, n3b):
    g = N // _T_PRE
    row = pl.BlockSpec((_T_PRE, C), lambda i: (i, 0))
    full = lambda a, b: pl.BlockSpec((a, b), lambda i: (0, 0))
    return pl.pallas_call(
        _post_body,
        grid=(g,),
        in_specs=[row, row, row,
                  full(1, C), full(1, C),
                  full(C, C), full(C, C), full(1, C),
                  full(1, C), full(1, C),
                  full(C, C), full(1, C), full(1, C), full(1, C),
                  full(C, C), full(1, C), full(1, C)],
        out_specs=row,
        out_shape=jax.ShapeDtypeStruct((N, C), jnp.float32),
    )(fm, cb, feat, nag, nab, wla1a, wla1b, bla1, lag, lab, wla2, bla2,
      n2g, n2b, wfc3, n3g, n3b)


def kernel(feat, coord, reference_index, gauss_dist, params):
    p = params
    mp = p['mamba']
    r1 = lambda a: a.reshape(1, -1)

    at = (-jnp.exp(mp['A_log'])).T
    wxdt = mp['W_x'][:, :DT_RANK] @ mp['W_dt']
    wxb = mp['W_x'][:, DT_RANK:DT_RANK + D_STATE]
    wxc = mp['W_x'][:, DT_RANK + D_STATE:]
    convw = mp['conv_w'].T

    f, xc, z, delta, bm, cm = _pre_call(
        feat, p['W_fc1'], r1(p['n1_g']), r1(p['n1_b']), r1(mp['rms_w']),
        mp['W_in'], convw, r1(mp['conv_b']), wxdt, r1(mp['b_dt']),
        wxb, wxc)

    npadk = (_NPAD - N) * K + (_GH - _GL) * _G * K
    idx_flat = jnp.concatenate(
        [reference_index.reshape(-1), jnp.zeros((npadk,), jnp.int32)])
    gauss_flat = jnp.concatenate(
        [gauss_dist.reshape(-1), jnp.zeros((npadk,), jnp.float32)])
    comb = _gather_call(f, idx_flat, gauss_flat)[:N]

    fm = _scan_call(delta, bm, cm, xc, z, f, at, r1(mp['D']),
                    mp['W_out'], r1(p['na_g']), r1(p['na_b']))

    return _post_call(
        fm, comb, feat, r1(p['na_g']), r1(p['na_b']),
        p['W_la1'][:C], p['W_la1'][C:], r1(p['b_la1']),
        r1(p['la_g']), r1(p['la_b']), p['W_la2'], r1(p['b_la2']),
        r1(p['n2_g']), r1(p['n2_b']), p['W_fc3'], r1(p['n3_g']),
        r1(p['n3_b']))


# speedup vs baseline: 1.6271x; 1.2948x over previous
import functools

import jax
import jax.numpy as jnp
from jax import lax
from jax.experimental import pallas as pl
from jax.experimental.pallas import tpu as pltpu
from jax.experimental.pallas import tpu_sc as plsc

N, K, C = 10000, 32, 128
D_INNER, D_STATE, D_CONV, DT_RANK = 256, 16, 4, 8

_T_PRE = 1000
_T_SCAN = 200
_G = 4
_NW = 32
_NG = N // _G


def _ln(x, g, b):
    m = jnp.mean(x, axis=-1, keepdims=True)
    xc = x - m
    v = jnp.mean(xc * xc, axis=-1, keepdims=True)
    return xc * lax.rsqrt(v + 1e-5) * g + b


def _softplus(x):
    return jnp.maximum(x, 0.0) + jnp.log(1.0 + jnp.exp(-jnp.abs(x)))


def _silu(x):
    return x / (1.0 + jnp.exp(-x))


def _dot(a, b):
    return lax.dot_general(a, b, (((1,), (0,)), ((), ())),
                           preferred_element_type=jnp.float32)


def _pre_body(feat_ref, halo_ref, wfc1, n1g, n1b, rmsw, win, convw, convb,
              wxdt, bdt, wxb, wxc,
              f_out, xc_out, z_out, dl_out, b_out, c_out):
    i = pl.program_id(0)

    def fproj(x):
        return jnp.maximum(_ln(_dot(x, wfc1[...]), n1g[...], n1b[...]), 0.0)

    def rms(x):
        return x * lax.rsqrt(jnp.mean(x * x, axis=-1, keepdims=True) + 1e-5) \
            * rmsw[...]

    f = fproj(feat_ref[...])
    f_out[...] = f
    xz = _dot(rms(f), win[...])
    xp = xz[:, :D_INNER]
    z_out[...] = xz[:, D_INNER:]

    fh = fproj(halo_ref[...])
    xph = _dot(rms(fh), win[...])[:, :D_INNER]
    xph = jnp.where(i == 0, 0.0, xph)
    ext = jnp.concatenate([xph[8 - (D_CONV - 1):], xp], axis=0)
    s = ext[0:_T_PRE] * convw[0:1, :]
    for k in range(1, D_CONV):
        s = s + ext[k:k + _T_PRE] * convw[k:k + 1, :]
    xc = _silu(s + convb[...])
    xc_out[...] = xc
    dl_out[...] = _softplus(_dot(xc, wxdt[...]) + bdt[...])
    b_out[...] = _dot(xc, wxb[...])
    c_out[...] = _dot(xc, wxc[...])


def _pre_call(feat, wfc1, n1g, n1b, rmsw, win, convw, convb, wxdt, bdt,
              wxb, wxc):
    g = N // _T_PRE
    row = lambda w: pl.BlockSpec((_T_PRE, w), lambda i: (i, 0))
    full = lambda a, b: pl.BlockSpec((a, b), lambda i: (0, 0))
    halo = pl.BlockSpec((8, C), lambda i: (jnp.maximum(i * (_T_PRE // 8) - 1, 0), 0))
    out_shapes = (
        jax.ShapeDtypeStruct((N, C), jnp.float32),
        jax.ShapeDtypeStruct((N, D_INNER), jnp.float32),
        jax.ShapeDtypeStruct((N, D_INNER), jnp.float32),
        jax.ShapeDtypeStruct((N, D_INNER), jnp.float32),
        jax.ShapeDtypeStruct((N, D_STATE), jnp.float32),
        jax.ShapeDtypeStruct((N, D_STATE), jnp.float32),
    )
    return pl.pallas_call(
        _pre_body,
        grid=(g,),
        in_specs=[
            row(C), halo,
            full(C, C), full(1, C), full(1, C), full(1, C),
            full(C, 2 * D_INNER), full(D_CONV, D_INNER), full(1, D_INNER),
            full(D_INNER, D_INNER), full(1, D_INNER),
            full(D_INNER, D_STATE), full(D_INNER, D_STATE),
        ],
        out_specs=(row(C), row(D_INNER), row(D_INNER), row(D_INNER),
                   row(D_STATE), row(D_STATE)),
        out_shape=out_shapes,
    )(feat, feat, wfc1, n1g, n1b, rmsw, win, convw, convb, wxdt, bdt,
      wxb, wxc)


def _scan_body(dl_ref, b_ref, c_ref, xc_ref, z_ref, f_ref, at_ref, dvec_ref,
               wout_ref, nag_ref, nab_ref, fm_out, h_c, da_s, dbx_s, hist_s):
    i = pl.program_id(0)

    @pl.when(i == 0)
    def _():
        h_c[...] = jnp.zeros_like(h_c)

    dl = dl_ref[...]
    da_s[...] = jnp.exp(dl[:, None, :] * at_ref[...][None])
    dbx_s[...] = (dl * xc_ref[...])[:, None, :] * b_ref[...][:, :, None]

    def step(t, h):
        h = da_s[pl.ds(t, 1)] * h + dbx_s[pl.ds(t, 1)]
        hist_s[pl.ds(t, 1)] = h
        return h

    h = lax.fori_loop(0, _T_SCAN, step, h_c[...])
    h_c[...] = h

    ys = jnp.sum(hist_s[...] * c_ref[...][:, :, None], axis=1)
    y = (ys + xc_ref[...] * dvec_ref[...]) * _silu(z_ref[...])
    m = f_ref[...] + _dot(y, wout_ref[...])
    fm_out[...] = _ln(m, nag_ref[...], nab_ref[...])


def _scan_call(delta, bm, cm, xc, z, f, at, dvec, wout, nag, nab):
    g = N // _T_SCAN
    row = lambda w: pl.BlockSpec((_T_SCAN, w), lambda i: (i, 0))
    full = lambda a, b: pl.BlockSpec((a, b), lambda i: (0, 0))
    return pl.pallas_call(
        _scan_body,
        grid=(g,),
        in_specs=[row(D_INNER), row(D_STATE), row(D_STATE), row(D_INNER),
                  row(D_INNER), row(C),
                  full(D_STATE, D_INNER), full(1, D_INNER),
                  full(D_INNER, C), full(1, C), full(1, C)],
        out_specs=row(C),
        out_shape=jax.ShapeDtypeStruct((N, C), jnp.float32),
        scratch_shapes=[
            pltpu.VMEM((1, D_STATE, D_INNER), jnp.float32),
            pltpu.VMEM((_T_SCAN, D_STATE, D_INNER), jnp.float32),
            pltpu.VMEM((_T_SCAN, D_STATE, D_INNER), jnp.float32),
            pltpu.VMEM((_T_SCAN, D_STATE, D_INNER), jnp.float32),
        ],
    )(delta, bm, cm, xc, z, f, at, dvec, wout, nag, nab)


def _gather_body(f_hbm, idxf_hbm, gaussf_hbm, out_hbm, idx_v, w_v, rows_v,
                 acc_v, sem):
    wid = lax.axis_index("s") * 2 + lax.axis_index("c")
    n_my = (_NG - wid + _NW - 1) // _NW

    def body(gi, carry):
        grp = wid + gi * _NW
        base = grp * (_G * K)
        pltpu.sync_copy(idxf_hbm.at[pl.ds(base, _G * K)], idx_v)
        pltpu.sync_copy(gaussf_hbm.at[pl.ds(base, _G * K)], w_v)
        pltpu.async_copy(f_hbm.at[idx_v], rows_v, sem).wait()
        for r in range(_G):
            wvecs = [w_v[pl.ds(r * K + 16 * j, 16)] for j in range(K // 16)]
            ws = [wvecs[k // 16][k % 16] for k in range(K)]
            for c8 in range(C // 16):
                parts = [jnp.zeros((16,), jnp.float32) for _ in range(4)]
                for k in range(K):
                    v = rows_v[r * K + k, pl.ds(c8 * 16, 16)]
                    parts[k % 4] = parts[k % 4] + ws[k] * v
                acc_v[r, pl.ds(c8 * 16, 16)] = \
                    (parts[0] + parts[1]) + (parts[2] + parts[3])
        pltpu.sync_copy(acc_v, out_hbm.at[pl.ds(grp * _G, _G)])
        return carry

    lax.fori_loop(0, n_my, body, 0)


def _gather_call(f, idx_flat, gauss_flat):
    mesh = plsc.VectorSubcoreMesh(core_axis_name="c", subcore_axis_name="s")
    kfn = functools.partial(
        pl.kernel,
        mesh=mesh,
        out_type=jax.ShapeDtypeStruct((N, C), jnp.float32),
        scratch_types=[
            pltpu.VMEM((_G * K,), jnp.int32),
            pltpu.VMEM((_G * K,), jnp.float32),
            pltpu.VMEM((_G * K, C), jnp.float32),
            pltpu.VMEM((_G, C), jnp.float32),
            pltpu.SemaphoreType.DMA,
        ],
    )(_gather_body)
    return kfn(f, idx_flat, gauss_flat)


def _post_body(fm_ref, cb_ref, feat_ref, nag, nab, wla1a, wla1b, bla1,
               lag, lab, wla2, bla2, n2g, n2b, wfc3, n3g, n3b, out_ref):
    comb = _ln(cb_ref[...], nag[...], nab[...])
    h = _dot(fm_ref[...], wla1a[...]) + _dot(comb, wla1b[...]) + bla1[...]
    h = jnp.maximum(_ln(h, lag[...], lab[...]), 0.0)
    res = _dot(h, wla2[...]) + bla2[...]
    f2 = jnp.maximum(_ln(res, n2g[...], n2b[...]), 0.0)
    f3 = _ln(_dot(f2, wfc3[...]), n3g[...], n3b[...])
    out_ref[...] = jnp.maximum(feat_ref[...] + f3, 0.0)


def _post_call(fm, cb, feat, nag, nab, wla1a, wla1b, bla1, lag, lab,
               wla2, bla2, n2g, n2b, wfc3, n3g, n3b):
    g = N // _T_PRE
    row = pl.BlockSpec((_T_PRE, C), lambda i: (i, 0))
    full = lambda a, b: pl.BlockSpec((a, b), lambda i: (0, 0))
    return pl.pallas_call(
        _post_body,
        grid=(g,),
        in_specs=[row, row, row,
                  full(1, C), full(1, C),
                  full(C, C), full(C, C), full(1, C),
                  full(1, C), full(1, C),
                  full(C, C), full(1, C), full(1, C), full(1, C),
                  full(C, C), full(1, C), full(1, C)],
        out_specs=row,
        out_shape=jax.ShapeDtypeStruct((N, C), jnp.float32),
    )(fm, cb, feat, nag, nab, wla1a, wla1b, bla1, lag, lab, wla2, bla2,
      n2g, n2b, wfc3, n3g, n3b)


def kernel(feat, coord, reference_index, gauss_dist, params):
    p = params
    mp = p['mamba']
    r1 = lambda a: a.reshape(1, -1)

    at = (-jnp.exp(mp['A_log'])).T
    wxdt = mp['W_x'][:, :DT_RANK] @ mp['W_dt']
    wxb = mp['W_x'][:, DT_RANK:DT_RANK + D_STATE]
    wxc = mp['W_x'][:, DT_RANK + D_STATE:]
    convw = mp['conv_w'].T

    f, xc, z, delta, bm, cm = _pre_call(
        feat, p['W_fc1'], r1(p['n1_g']), r1(p['n1_b']), r1(mp['rms_w']),
        mp['W_in'], convw, r1(mp['conv_b']), wxdt, r1(mp['b_dt']),
        wxb, wxc)

    comb = _gather_call(f, reference_index.reshape(-1),
                        gauss_dist.reshape(-1))

    fm = _scan_call(delta, bm, cm, xc, z, f, at, r1(mp['D']),
                    mp['W_out'], r1(p['na_g']), r1(p['na_b']))

    return _post_call(
        fm, comb, feat, r1(p['na_g']), r1(p['na_b']),
        p['W_la1'][:C], p['W_la1'][C:], r1(p['b_la1']),
        r1(p['la_g']), r1(p['la_b']), p['W_la2'], r1(p['b_la2']),
        r1(p['n2_g']), r1(p['n2_b']), p['W_fc3'], r1(p['n3_g']),
        r1(p['n3_b']))


# speedup vs baseline: 1.7715x; 1.0887x over previous
import functools

import jax
import jax.numpy as jnp
from jax import lax
from jax.experimental import pallas as pl
from jax.experimental.pallas import tpu as pltpu
from jax.experimental.pallas import tpu_sc as plsc

N, K, C = 10000, 32, 128
D_INNER, D_STATE, D_CONV, DT_RANK = 256, 16, 4, 8

_T_PRE = 1000
_T_SCAN = 200
_G = 4
_NW = 32
_NG = N // _G


def _ln(x, g, b):
    m = jnp.mean(x, axis=-1, keepdims=True)
    xc = x - m
    v = jnp.mean(xc * xc, axis=-1, keepdims=True)
    return xc * lax.rsqrt(v + 1e-5) * g + b


def _softplus(x):
    return jnp.maximum(x, 0.0) + jnp.log(1.0 + jnp.exp(-jnp.abs(x)))


def _silu(x):
    return x / (1.0 + jnp.exp(-x))


def _dot(a, b):
    return lax.dot_general(a, b, (((1,), (0,)), ((), ())),
                           preferred_element_type=jnp.float32)


def _pre_body(feat_ref, halo_ref, wfc1, n1g, n1b, rmsw, win, convw, convb,
              wxdt, bdt, wxb, wxc,
              f_out, xc_out, z_out, dl_out, b_out, c_out):
    i = pl.program_id(0)

    def fproj(x):
        return jnp.maximum(_ln(_dot(x, wfc1[...]), n1g[...], n1b[...]), 0.0)

    def rms(x):
        return x * lax.rsqrt(jnp.mean(x * x, axis=-1, keepdims=True) + 1e-5) \
            * rmsw[...]

    f = fproj(feat_ref[...])
    f_out[...] = f
    xz = _dot(rms(f), win[...])
    xp = xz[:, :D_INNER]
    z_out[...] = xz[:, D_INNER:]

    fh = fproj(halo_ref[...])
    xph = _dot(rms(fh), win[...])[:, :D_INNER]
    xph = jnp.where(i == 0, 0.0, xph)
    ext = jnp.concatenate([xph[8 - (D_CONV - 1):], xp], axis=0)
    s = ext[0:_T_PRE] * convw[0:1, :]
    for k in range(1, D_CONV):
        s = s + ext[k:k + _T_PRE] * convw[k:k + 1, :]
    xc = _silu(s + convb[...])
    xc_out[...] = xc
    dl_out[...] = _softplus(_dot(xc, wxdt[...]) + bdt[...])
    b_out[...] = _dot(xc, wxb[...])
    c_out[...] = _dot(xc, wxc[...])


def _pre_call(feat, wfc1, n1g, n1b, rmsw, win, convw, convb, wxdt, bdt,
              wxb, wxc):
    g = N // _T_PRE
    row = lambda w: pl.BlockSpec((_T_PRE, w), lambda i: (i, 0))
    full = lambda a, b: pl.BlockSpec((a, b), lambda i: (0, 0))
    halo = pl.BlockSpec((8, C), lambda i: (jnp.maximum(i * (_T_PRE // 8) - 1, 0), 0))
    out_shapes = (
        jax.ShapeDtypeStruct((N, C), jnp.float32),
        jax.ShapeDtypeStruct((N, D_INNER), jnp.float32),
        jax.ShapeDtypeStruct((N, D_INNER), jnp.float32),
        jax.ShapeDtypeStruct((N, D_INNER), jnp.float32),
        jax.ShapeDtypeStruct((N, D_STATE), jnp.float32),
        jax.ShapeDtypeStruct((N, D_STATE), jnp.float32),
    )
    return pl.pallas_call(
        _pre_body,
        grid=(g,),
        in_specs=[
            row(C), halo,
            full(C, C), full(1, C), full(1, C), full(1, C),
            full(C, 2 * D_INNER), full(D_CONV, D_INNER), full(1, D_INNER),
            full(D_INNER, D_INNER), full(1, D_INNER),
            full(D_INNER, D_STATE), full(D_INNER, D_STATE),
        ],
        out_specs=(row(C), row(D_INNER), row(D_INNER), row(D_INNER),
                   row(D_STATE), row(D_STATE)),
        out_shape=out_shapes,
    )(feat, feat, wfc1, n1g, n1b, rmsw, win, convw, convb, wxdt, bdt,
      wxb, wxc)


def _scan_body(dl_ref, b_ref, c_ref, xc_ref, z_ref, f_ref, at_ref, dvec_ref,
               wout_ref, nag_ref, nab_ref, fm_out, h_c, da_s, dbx_s, hist_s):
    i = pl.program_id(0)

    @pl.when(i == 0)
    def _():
        h_c[...] = jnp.zeros_like(h_c)

    dl = dl_ref[...]
    da_s[...] = jnp.exp(dl[:, None, :] * at_ref[...][None])
    dbx_s[...] = (dl * xc_ref[...])[:, None, :] * b_ref[...][:, :, None]

    def step(t, h):
        h = da_s[pl.ds(t, 1)] * h + dbx_s[pl.ds(t, 1)]
        hist_s[pl.ds(t, 1)] = h
        return h

    h = lax.fori_loop(0, _T_SCAN, step, h_c[...])
    h_c[...] = h

    ys = jnp.sum(hist_s[...] * c_ref[...][:, :, None], axis=1)
    y = (ys + xc_ref[...] * dvec_ref[...]) * _silu(z_ref[...])
    m = f_ref[...] + _dot(y, wout_ref[...])
    fm_out[...] = _ln(m, nag_ref[...], nab_ref[...])


def _scan_call(delta, bm, cm, xc, z, f, at, dvec, wout, nag, nab):
    g = N // _T_SCAN
    row = lambda w: pl.BlockSpec((_T_SCAN, w), lambda i: (i, 0))
    full = lambda a, b: pl.BlockSpec((a, b), lambda i: (0, 0))
    return pl.pallas_call(
        _scan_body,
        grid=(g,),
        in_specs=[row(D_INNER), row(D_STATE), row(D_STATE), row(D_INNER),
                  row(D_INNER), row(C),
                  full(D_STATE, D_INNER), full(1, D_INNER),
                  full(D_INNER, C), full(1, C), full(1, C)],
        out_specs=row(C),
        out_shape=jax.ShapeDtypeStruct((N, C), jnp.float32),
        scratch_shapes=[
            pltpu.VMEM((1, D_STATE, D_INNER), jnp.float32),
            pltpu.VMEM((_T_SCAN, D_STATE, D_INNER), jnp.float32),
            pltpu.VMEM((_T_SCAN, D_STATE, D_INNER), jnp.float32),
            pltpu.VMEM((_T_SCAN, D_STATE, D_INNER), jnp.float32),
        ],
    )(delta, bm, cm, xc, z, f, at, dvec, wout, nag, nab)


def _gather_body(f_hbm, idxf_hbm, gaussf_hbm, out_hbm, idx_v, w_v, rows_v,
                 acc_v, f_sh, sem):
    s = lax.axis_index("s")
    wid = s * 2 + lax.axis_index("c")
    n_my = (_NG - wid + _NW - 1) // _NW

    @pl.when(s < 15)
    def _():
        pltpu.sync_copy(f_hbm.at[pl.ds(s * 632, 632)],
                        f_sh.at[pl.ds(s * 632, 632)])

    @pl.when(s == 15)
    def _():
        pltpu.sync_copy(f_hbm.at[pl.ds(9480, N - 9480)],
                        f_sh.at[pl.ds(9480, N - 9480)])

    plsc.subcore_barrier()

    def body(gi, carry):
        grp = wid + gi * _NW
        base = grp * (_G * K)
        pltpu.sync_copy(idxf_hbm.at[pl.ds(base, _G * K)], idx_v)
        pltpu.sync_copy(gaussf_hbm.at[pl.ds(base, _G * K)], w_v)
        pltpu.async_copy(f_sh.at[idx_v], rows_v, sem).wait()
        for r in range(_G):
            wvecs = [w_v[pl.ds(r * K + 16 * j, 16)] for j in range(K // 16)]
            ws = [wvecs[k // 16][k % 16] for k in range(K)]
            for c8 in range(C // 16):
                parts = [jnp.zeros((16,), jnp.float32) for _ in range(4)]
                for k in range(K):
                    v = rows_v[r * K + k, pl.ds(c8 * 16, 16)]
                    parts[k % 4] = parts[k % 4] + ws[k] * v
                acc_v[r, pl.ds(c8 * 16, 16)] = \
                    (parts[0] + parts[1]) + (parts[2] + parts[3])
        pltpu.sync_copy(acc_v, out_hbm.at[pl.ds(grp * _G, _G)])
        return carry

    lax.fori_loop(0, n_my, body, 0)


def _gather_call(f, idx_flat, gauss_flat):
    mesh = plsc.VectorSubcoreMesh(core_axis_name="c", subcore_axis_name="s")
    kfn = functools.partial(
        pl.kernel,
        mesh=mesh,
        out_type=jax.ShapeDtypeStruct((N, C), jnp.float32),
        scratch_types=[
            pltpu.VMEM((_G * K,), jnp.int32),
            pltpu.VMEM((_G * K,), jnp.float32),
            pltpu.VMEM((_G * K, C), jnp.float32),
            pltpu.VMEM((_G, C), jnp.float32),
            pltpu.VMEM_SHARED((N, C), jnp.float32),
            pltpu.SemaphoreType.DMA,
        ],
    )(_gather_body)
    return kfn(f, idx_flat, gauss_flat)


def _post_body(fm_ref, cb_ref, feat_ref, nag, nab, wla1a, wla1b, bla1,
               lag, lab, wla2, bla2, n2g, n2b, wfc3, n3g, n3b, out_ref):
    comb = _ln(cb_ref[...], nag[...], nab[...])
    h = _dot(fm_ref[...], wla1a[...]) + _dot(comb, wla1b[...]) + bla1[...]
    h = jnp.maximum(_ln(h, lag[...], lab[...]), 0.0)
    res = _dot(h, wla2[...]) + bla2[...]
    f2 = jnp.maximum(_ln(res, n2g[...], n2b[...]), 0.0)
    f3 = _ln(_dot(f2, wfc3[...]), n3g[...], n3b[...])
    out_ref[...] = jnp.maximum(feat_ref[...] + f3, 0.0)


def _post_call(fm, cb, feat, nag, nab, wla1a, wla1b, bla1, lag, lab,
               wla2, bla2, n2g, n2b, wfc3, n3g, n3b):
    g = N // _T_PRE
    row = pl.BlockSpec((_T_PRE, C), lambda i: (i, 0))
    full = lambda a, b: pl.BlockSpec((a, b), lambda i: (0, 0))
    return pl.pallas_call(
        _post_body,
        grid=(g,),
        in_specs=[row, row, row,
                  full(1, C), full(1, C),
                  full(C, C), full(C, C), full(1, C),
                  full(1, C), full(1, C),
                  full(C, C), full(1, C), full(1, C), full(1, C),
                  full(C, C), full(1, C), full(1, C)],
        out_specs=row,
        out_shape=jax.ShapeDtypeStruct((N, C), jnp.float32),
    )(fm, cb, feat, nag, nab, wla1a, wla1b, bla1, lag, lab, wla2, bla2,
      n2g, n2b, wfc3, n3g, n3b)


def kernel(feat, coord, reference_index, gauss_dist, params):
    p = params
    mp = p['mamba']
    r1 = lambda a: a.reshape(1, -1)

    at = (-jnp.exp(mp['A_log'])).T
    wxdt = mp['W_x'][:, :DT_RANK] @ mp['W_dt']
    wxb = mp['W_x'][:, DT_RANK:DT_RANK + D_STATE]
    wxc = mp['W_x'][:, DT_RANK + D_STATE:]
    convw = mp['conv_w'].T

    f, xc, z, delta, bm, cm = _pre_call(
        feat, p['W_fc1'], r1(p['n1_g']), r1(p['n1_b']), r1(mp['rms_w']),
        mp['W_in'], convw, r1(mp['conv_b']), wxdt, r1(mp['b_dt']),
        wxb, wxc)

    comb = _gather_call(f, reference_index.reshape(-1),
                        gauss_dist.reshape(-1))

    fm = _scan_call(delta, bm, cm, xc, z, f, at, r1(mp['D']),
                    mp['W_out'], r1(p['na_g']), r1(p['na_b']))

    return _post_call(
        fm, comb, feat, r1(p['na_g']), r1(p['na_b']),
        p['W_la1'][:C], p['W_la1'][C:], r1(p['b_la1']),
        r1(p['la_g']), r1(p['la_b']), p['W_la2'], r1(p['b_la2']),
        r1(p['n2_g']), r1(p['n2_b']), p['W_fc3'], r1(p['n3_g']),
        r1(p['n3_b']))


# speedup vs baseline: 2.9343x; 1.6564x over previous
import functools

import jax
import jax.numpy as jnp
from jax import lax
from jax.experimental import pallas as pl
from jax.experimental.pallas import tpu as pltpu
from jax.experimental.pallas import tpu_sc as plsc

N, K, C = 10000, 32, 128
D_INNER, D_STATE, D_CONV, DT_RANK = 256, 16, 4, 8

_T_PRE = 1000
_T_SCAN = 200
_G = 2
_NW = 32
_GPW = 160
_NPAD = _NW * _GPW * _G


def _ln(x, g, b):
    m = jnp.mean(x, axis=-1, keepdims=True)
    xc = x - m
    v = jnp.mean(xc * xc, axis=-1, keepdims=True)
    return xc * lax.rsqrt(v + 1e-5) * g + b


def _softplus(x):
    return jnp.maximum(x, 0.0) + jnp.log(1.0 + jnp.exp(-jnp.abs(x)))


def _silu(x):
    return x / (1.0 + jnp.exp(-x))


def _dot(a, b):
    return lax.dot_general(a, b, (((1,), (0,)), ((), ())),
                           preferred_element_type=jnp.float32)


def _pre_body(feat_ref, halo_ref, wfc1, n1g, n1b, rmsw, win, convw, convb,
              wxdt, bdt, wxb, wxc,
              f_out, xc_out, z_out, dl_out, b_out, c_out):
    i = pl.program_id(0)

    def fproj(x):
        return jnp.maximum(_ln(_dot(x, wfc1[...]), n1g[...], n1b[...]), 0.0)

    def rms(x):
        return x * lax.rsqrt(jnp.mean(x * x, axis=-1, keepdims=True) + 1e-5) \
            * rmsw[...]

    f = fproj(feat_ref[...])
    f_out[...] = f
    xz = _dot(rms(f), win[...])
    xp = xz[:, :D_INNER]
    z_out[...] = xz[:, D_INNER:]

    fh = fproj(halo_ref[...])
    xph = _dot(rms(fh), win[...])[:, :D_INNER]
    xph = jnp.where(i == 0, 0.0, xph)
    ext = jnp.concatenate([xph[8 - (D_CONV - 1):], xp], axis=0)
    s = ext[0:_T_PRE] * convw[0:1, :]
    for k in range(1, D_CONV):
        s = s + ext[k:k + _T_PRE] * convw[k:k + 1, :]
    xc = _silu(s + convb[...])
    xc_out[...] = xc
    dl_out[...] = _softplus(_dot(xc, wxdt[...]) + bdt[...])
    b_out[...] = _dot(xc, wxb[...])
    c_out[...] = _dot(xc, wxc[...])


def _pre_call(feat, wfc1, n1g, n1b, rmsw, win, convw, convb, wxdt, bdt,
              wxb, wxc):
    g = N // _T_PRE
    row = lambda w: pl.BlockSpec((_T_PRE, w), lambda i: (i, 0))
    full = lambda a, b: pl.BlockSpec((a, b), lambda i: (0, 0))
    halo = pl.BlockSpec((8, C), lambda i: (jnp.maximum(i * (_T_PRE // 8) - 1, 0), 0))
    out_shapes = (
        jax.ShapeDtypeStruct((N, C), jnp.float32),
        jax.ShapeDtypeStruct((N, D_INNER), jnp.float32),
        jax.ShapeDtypeStruct((N, D_INNER), jnp.float32),
        jax.ShapeDtypeStruct((N, D_INNER), jnp.float32),
        jax.ShapeDtypeStruct((N, D_STATE), jnp.float32),
        jax.ShapeDtypeStruct((N, D_STATE), jnp.float32),
    )
    return pl.pallas_call(
        _pre_body,
        grid=(g,),
        in_specs=[
            row(C), halo,
            full(C, C), full(1, C), full(1, C), full(1, C),
            full(C, 2 * D_INNER), full(D_CONV, D_INNER), full(1, D_INNER),
            full(D_INNER, D_INNER), full(1, D_INNER),
            full(D_INNER, D_STATE), full(D_INNER, D_STATE),
        ],
        out_specs=(row(C), row(D_INNER), row(D_INNER), row(D_INNER),
                   row(D_STATE), row(D_STATE)),
        out_shape=out_shapes,
    )(feat, feat, wfc1, n1g, n1b, rmsw, win, convw, convb, wxdt, bdt,
      wxb, wxc)


def _scan_body(dl_ref, b_ref, c_ref, xc_ref, z_ref, f_ref, at_ref, dvec_ref,
               wout_ref, nag_ref, nab_ref, fm_out, h_c, da_s, dbx_s, hist_s):
    i = pl.program_id(0)

    @pl.when(i == 0)
    def _():
        h_c[...] = jnp.zeros_like(h_c)

    dl = dl_ref[...]
    da_s[...] = jnp.exp(dl[:, None, :] * at_ref[...][None])
    dbx_s[...] = (dl * xc_ref[...])[:, None, :] * b_ref[...][:, :, None]

    def step(t, h):
        h = da_s[pl.ds(t, 1)] * h + dbx_s[pl.ds(t, 1)]
        hist_s[pl.ds(t, 1)] = h
        return h

    h = lax.fori_loop(0, _T_SCAN, step, h_c[...])
    h_c[...] = h

    ys = jnp.sum(hist_s[...] * c_ref[...][:, :, None], axis=1)
    y = (ys + xc_ref[...] * dvec_ref[...]) * _silu(z_ref[...])
    m = f_ref[...] + _dot(y, wout_ref[...])
    fm_out[...] = _ln(m, nag_ref[...], nab_ref[...])


def _scan_call(delta, bm, cm, xc, z, f, at, dvec, wout, nag, nab):
    g = N // _T_SCAN
    row = lambda w: pl.BlockSpec((_T_SCAN, w), lambda i: (i, 0))
    full = lambda a, b: pl.BlockSpec((a, b), lambda i: (0, 0))
    return pl.pallas_call(
        _scan_body,
        grid=(g,),
        in_specs=[row(D_INNER), row(D_STATE), row(D_STATE), row(D_INNER),
                  row(D_INNER), row(C),
                  full(D_STATE, D_INNER), full(1, D_INNER),
                  full(D_INNER, C), full(1, C), full(1, C)],
        out_specs=row(C),
        out_shape=jax.ShapeDtypeStruct((N, C), jnp.float32),
        scratch_shapes=[
            pltpu.VMEM((1, D_STATE, D_INNER), jnp.float32),
            pltpu.VMEM((_T_SCAN, D_STATE, D_INNER), jnp.float32),
            pltpu.VMEM((_T_SCAN, D_STATE, D_INNER), jnp.float32),
            pltpu.VMEM((_T_SCAN, D_STATE, D_INNER), jnp.float32),
        ],
    )(delta, bm, cm, xc, z, f, at, dvec, wout, nag, nab)


def _gather_body(f_hbm, idxf_hbm, gaussf_hbm, out_hbm, idx_v, w_v, rows_v,
                 acc_v, f_sh, lsem, sem):
    s = lax.axis_index("s")
    wid = s * 2 + lax.axis_index("c")

    @pl.when(s < 15)
    def _():
        pltpu.sync_copy(f_hbm.at[pl.ds(s * 632, 632)],
                        f_sh.at[pl.ds(s * 632, 632)])

    @pl.when(s == 15)
    def _():
        pltpu.sync_copy(f_hbm.at[pl.ds(9480, N - 9480)],
                        f_sh.at[pl.ds(9480, N - 9480)])

    gk = _G * K
    base = wid * (_GPW * gk)
    cp_i = pltpu.async_copy(idxf_hbm.at[pl.ds(base, _GPW * gk)], idx_v, lsem)
    cp_w = pltpu.async_copy(gaussf_hbm.at[pl.ds(base, _GPW * gk)], w_v, lsem)
    cp_i.wait()
    cp_w.wait()
    plsc.subcore_barrier()

    def fire(gi, buf):
        idx_sl = idx_v.at[pl.ds(gi * gk, gk)]
        return pltpu.async_copy(f_sh.at[idx_sl], rows_v.at[buf], sem)

    fire(0, 0)
    fire(1, 1)

    def body(gi, carry):
        buf = lax.rem(gi, 2)
        pltpu.make_async_copy(f_sh.at[idx_v.at[pl.ds(0, gk)]],
                              rows_v.at[buf], sem).wait()
        for r in range(_G):
            woff = gi * gk + r * K
            wvecs = [w_v[pl.ds(woff + 16 * j, 16)] for j in range(K // 16)]
            ws = [wvecs[k // 16][k % 16] for k in range(K)]
            for c8 in range(C // 16):
                parts = [jnp.zeros((16,), jnp.float32) for _ in range(4)]
                for k in range(K):
                    v = rows_v[buf, r * K + k, pl.ds(c8 * 16, 16)]
                    parts[k % 4] = parts[k % 4] + ws[k] * v
                acc_v[r, pl.ds(c8 * 16, 16)] = \
                    (parts[0] + parts[1]) + (parts[2] + parts[3])
        pltpu.sync_copy(acc_v, out_hbm.at[pl.ds((wid * _GPW + gi) * _G, _G)])
        @pl.when(gi + 2 < _GPW)
        def _():
            fire(gi + 2, buf)
        return carry

    lax.fori_loop(0, _GPW, body, 0)


def _gather_call(f, idx_flat, gauss_flat):
    mesh = plsc.VectorSubcoreMesh(core_axis_name="c", subcore_axis_name="s")
    kfn = functools.partial(
        pl.kernel,
        mesh=mesh,
        out_type=jax.ShapeDtypeStruct((_NPAD, C), jnp.float32),
        scratch_types=[
            pltpu.VMEM((_GPW * _G * K,), jnp.int32),
            pltpu.VMEM((_GPW * _G * K,), jnp.float32),
            pltpu.VMEM((2, _G * K, C), jnp.float32),
            pltpu.VMEM((_G, C), jnp.float32),
            pltpu.VMEM_SHARED((N, C), jnp.float32),
            pltpu.SemaphoreType.DMA,
            pltpu.SemaphoreType.DMA,
        ],
    )(_gather_body)
    return kfn(f, idx_flat, gauss_flat)


def _post_body(fm_ref, cb_ref, feat_ref, nag, nab, wla1a, wla1b, bla1,
               lag, lab, wla2, bla2, n2g, n2b, wfc3, n3g, n3b, out_ref):
    comb = _ln(cb_ref[...], nag[...], nab[...])
    h = _dot(fm_ref[...], wla1a[...]) + _dot(comb, wla1b[...]) + bla1[...]
    h = jnp.maximum(_ln(h, lag[...], lab[...]), 0.0)
    res = _dot(h, wla2[...]) + bla2[...]
    f2 = jnp.maximum(_ln(res, n2g[...], n2b[...]), 0.0)
    f3 = _ln(_dot(f2, wfc3[...]), n3g[...], n3b[...])
    out_ref[...] = jnp.maximum(feat_ref[...] + f3, 0.0)


def _post_call(fm, cb, feat, nag, nab, wla1a, wla1b, bla1, lag, lab,
               wla2, bla2, n2g, n2b, wfc3, n3g, n3b):
    g = N // _T_PRE
    row = pl.BlockSpec((_T_PRE, C), lambda i: (i, 0))
    full = lambda a, b: pl.BlockSpec((a, b), lambda i: (0, 0))
    return pl.pallas_call(
        _post_body,
        grid=(g,),
        in_specs=[row, row, row,
                  full(1, C), full(1, C),
                  full(C, C), full(C, C), full(1, C),
                  full(1, C), full(1, C),
                  full(C, C), full(1, C), full(1, C), full(1, C),
                  full(C, C), full(1, C), full(1, C)],
        out_specs=row,
        out_shape=jax.ShapeDtypeStruct((N, C), jnp.float32),
    )(fm, cb, feat, nag, nab, wla1a, wla1b, bla1, lag, lab, wla2, bla2,
      n2g, n2b, wfc3, n3g, n3b)


def kernel(feat, coord, reference_index, gauss_dist, params):
    p = params
    mp = p['mamba']
    r1 = lambda a: a.reshape(1, -1)

    at = (-jnp.exp(mp['A_log'])).T
    wxdt = mp['W_x'][:, :DT_RANK] @ mp['W_dt']
    wxb = mp['W_x'][:, DT_RANK:DT_RANK + D_STATE]
    wxc = mp['W_x'][:, DT_RANK + D_STATE:]
    convw = mp['conv_w'].T

    f, xc, z, delta, bm, cm = _pre_call(
        feat, p['W_fc1'], r1(p['n1_g']), r1(p['n1_b']), r1(mp['rms_w']),
        mp['W_in'], convw, r1(mp['conv_b']), wxdt, r1(mp['b_dt']),
        wxb, wxc)

    npadk = (_NPAD - N) * K
    idx_flat = jnp.concatenate(
        [reference_index.reshape(-1), jnp.zeros((npadk,), jnp.int32)])
    gauss_flat = jnp.concatenate(
        [gauss_dist.reshape(-1), jnp.zeros((npadk,), jnp.float32)])
    comb = _gather_call(f, idx_flat, gauss_flat)[:N]

    fm = _scan_call(delta, bm, cm, xc, z, f, at, r1(mp['D']),
                    mp['W_out'], r1(p['na_g']), r1(p['na_b']))

    return _post_call(
        fm, comb, feat, r1(p['na_g']), r1(p['na_b']),
        p['W_la1'][:C], p['W_la1'][C:], r1(p['b_la1']),
        r1(p['la_g']), r1(p['la_b']), p['W_la2'], r1(p['b_la2']),
        r1(p['n2_g']), r1(p['n2_b']), p['W_fc3'], r1(p['n3_g']),
        r1(p['n3_b']))
